# baseline (speedup 1.0000x reference)
import functools

import jax
import jax.numpy as jnp
from jax import lax
from jax.experimental import pallas as pl
from jax.experimental.pallas import tpu as pltpu
from jax.experimental.pallas import tpu_sc as plsc

_VOCAB = 1000000
_DIM = 64
_BATCH = 16384
_HIST = 200
_TOTAL = _BATCH * _HIST
_NW = 32
_PER_W = _TOTAL // _NW
_SUB = 128
_K = 8
_CHUNK = _SUB * _K
_NITER = _PER_W // _CHUNK

_mesh = plsc.VectorSubcoreMesh(core_axis_name="c", subcore_axis_name="s")


@functools.partial(
    pl.kernel,
    mesh=_mesh,
    out_type=jax.ShapeDtypeStruct((_TOTAL, _DIM), jnp.float32),
    scratch_types=[
        pltpu.VMEM((_K, _SUB), jnp.int32),
        pltpu.VMEM((_CHUNK, _DIM), jnp.float32),
        pltpu.SemaphoreType.DMA,
    ],
    compiler_params=pltpu.CompilerParams(use_tc_tiling_on_sc=False),
)
def _embed_gather(idx_hbm, table_hbm, out_hbm, idx_v, rows_v, sem):
    wid = lax.axis_index("s") * 2 + lax.axis_index("c")
    row_base = wid * _PER_W
    idx_base = row_base // _SUB

    def body(i, carry):
        ioff = pl.multiple_of(idx_base + i * _K, 8)
        pltpu.sync_copy(idx_hbm.at[pl.ds(ioff, _K)], idx_v)
        copies = []
        for j in range(_K):
            copies.append(
                pltpu.async_copy(
                    table_hbm.at[idx_v.at[j]],
                    rows_v.at[pl.ds(j * _SUB, _SUB)],
                    sem,
                )
            )
        for c in copies:
            c.wait()
        ooff = pl.multiple_of(row_base + i * _CHUNK, 8)
        pltpu.sync_copy(rows_v, out_hbm.at[pl.ds(ooff, _CHUNK)])
        return carry

    lax.fori_loop(0, _NITER, body, 0)


def kernel(x, table):
    idx2d = x.reshape(_TOTAL // _SUB, _SUB)
    out = _embed_gather(idx2d, table)
    return out.reshape(_BATCH, _HIST, _DIM)


# speedup vs baseline: 1.0246x; 1.0246x over previous
import functools

import jax
import jax.numpy as jnp
from jax import lax
from jax.experimental import pallas as pl
from jax.experimental.pallas import tpu as pltpu
from jax.experimental.pallas import tpu_sc as plsc

_VOCAB = 1000000
_DIM = 64
_BATCH = 16384
_HIST = 200
_TOTAL = _BATCH * _HIST
_NW = 32
_PER_W = _TOTAL // _NW
_SUB = 128
_K = 8
_CHUNK = _SUB * _K
_HALF = _CHUNK // 2
_NITER = _PER_W // _CHUNK

_mesh = plsc.VectorSubcoreMesh(core_axis_name="c", subcore_axis_name="s")


@functools.partial(
    pl.kernel,
    mesh=_mesh,
    out_type=jax.ShapeDtypeStruct((_TOTAL, _DIM), jnp.float32),
    scratch_types=[
        pltpu.VMEM((_K, _SUB), jnp.int32),
        pltpu.VMEM((_HALF, _DIM), jnp.float32),
        pltpu.VMEM((_HALF, _DIM), jnp.float32),
        pltpu.SemaphoreType.DMA,
        pltpu.SemaphoreType.DMA,
        pltpu.SemaphoreType.DMA,
        pltpu.SemaphoreType.DMA,
    ],
    compiler_params=pltpu.CompilerParams(use_tc_tiling_on_sc=False),
)
def _embed_gather(idx_hbm, table_hbm, out_hbm, idx_v, rows_a, rows_b,
                  sem_ga, sem_gb, sem_wa, sem_wb):
    wid = lax.axis_index("s") * 2 + lax.axis_index("c")
    row_base = wid * _PER_W
    idx_base = row_base // _SUB

    def body(g, carry):
        ioff = pl.multiple_of(idx_base + g * _K, 8)
        pltpu.sync_copy(idx_hbm.at[pl.ds(ioff, _K)], idx_v)

        ooff_a = pl.multiple_of(row_base + g * _CHUNK, 8)
        ooff_b = pl.multiple_of(row_base + g * _CHUNK + _HALF, 8)
        out_a = out_hbm.at[pl.ds(ooff_a, _HALF)]
        out_b = out_hbm.at[pl.ds(ooff_b, _HALF)]

        @pl.when(g > 0)
        def _():
            pltpu.make_async_copy(rows_a, out_a, sem_wa).wait()

        ga = [
            pltpu.async_copy(
                table_hbm.at[idx_v.at[j]],
                rows_a.at[pl.ds(j * _SUB, _SUB)],
                sem_ga,
            )
            for j in range(_K // 2)
        ]

        @pl.when(g > 0)
        def _():
            pltpu.make_async_copy(rows_b, out_b, sem_wb).wait()

        gb = [
            pltpu.async_copy(
                table_hbm.at[idx_v.at[j]],
                rows_b.at[pl.ds((j - _K // 2) * _SUB, _SUB)],
                sem_gb,
            )
            for j in range(_K // 2, _K)
        ]

        for c in ga:
            c.wait()
        pltpu.async_copy(rows_a, out_a, sem_wa)
        for c in gb:
            c.wait()
        pltpu.async_copy(rows_b, out_b, sem_wb)
        return carry

    lax.fori_loop(0, _NITER, body, 0)

    last_a = pl.multiple_of(row_base + (_NITER - 1) * _CHUNK, 8)
    last_b = pl.multiple_of(row_base + (_NITER - 1) * _CHUNK + _HALF, 8)
    pltpu.make_async_copy(rows_a, out_hbm.at[pl.ds(last_a, _HALF)], sem_wa).wait()
    pltpu.make_async_copy(rows_b, out_hbm.at[pl.ds(last_b, _HALF)], sem_wb).wait()


def kernel(x, table):
    idx2d = x.reshape(_TOTAL // _SUB, _SUB)
    out = _embed_gather(idx2d, table)
    return out.reshape(_BATCH, _HIST, _DIM)


# speedup vs baseline: 1.0255x; 1.0009x over previous
import functools

import jax
import jax.numpy as jnp
from jax import lax
from jax.experimental import pallas as pl
from jax.experimental.pallas import tpu as pltpu
from jax.experimental.pallas import tpu_sc as plsc

_VOCAB = 1000000
_DIM = 64
_BATCH = 16384
_HIST = 200
_NW = 32
_BPW = _BATCH // _NW
_NB = 4
_SPLITS = ((0, 104), (104, 96))
_NITER = _BPW // (2 * _NB)

_mesh = plsc.VectorSubcoreMesh(core_axis_name="c", subcore_axis_name="s")


@functools.partial(
    pl.kernel,
    mesh=_mesh,
    out_type=jax.ShapeDtypeStruct((_BATCH, _HIST, _DIM), jnp.float32),
    scratch_types=[
        pltpu.VMEM((2 * _NB, _HIST), jnp.int32),
        pltpu.VMEM((_NB, _HIST, _DIM), jnp.float32),
        pltpu.VMEM((_NB, _HIST, _DIM), jnp.float32),
        pltpu.SemaphoreType.DMA,
        pltpu.SemaphoreType.DMA,
        pltpu.SemaphoreType.DMA,
        pltpu.SemaphoreType.DMA,
    ],
    compiler_params=pltpu.CompilerParams(use_tc_tiling_on_sc=False),
)
def _embed_gather(x_hbm, table_hbm, out_hbm, idx_v, rows_a, rows_b,
                  sem_ga, sem_gb, sem_wa, sem_wb):
    wid = lax.axis_index("s") * 2 + lax.axis_index("c")
    b_base = wid * _BPW

    def fire_gathers(rows_buf, j0, sem):
        copies = []
        for j in range(_NB):
            for h, w in _SPLITS:
                copies.append(
                    pltpu.async_copy(
                        table_hbm.at[idx_v.at[j0 + j, pl.ds(h, w)]],
                        rows_buf.at[j, pl.ds(h, w)],
                        sem,
                    )
                )
        return copies

    def body(g, carry):
        b0 = b_base + g * 2 * _NB
        pltpu.sync_copy(x_hbm.at[pl.ds(b0, 2 * _NB)], idx_v)

        out_a = out_hbm.at[pl.ds(b0, _NB)]
        out_b = out_hbm.at[pl.ds(b0 + _NB, _NB)]

        @pl.when(g > 0)
        def _():
            pltpu.make_async_copy(rows_a, out_a, sem_wa).wait()

        ga = fire_gathers(rows_a, 0, sem_ga)

        @pl.when(g > 0)
        def _():
            pltpu.make_async_copy(rows_b, out_b, sem_wb).wait()

        gb = fire_gathers(rows_b, _NB, sem_gb)

        for c in ga:
            c.wait()
        pltpu.async_copy(rows_a, out_a, sem_wa)
        for c in gb:
            c.wait()
        pltpu.async_copy(rows_b, out_b, sem_wb)
        return carry

    lax.fori_loop(0, _NITER, body, 0)

    last = b_base + (_NITER - 1) * 2 * _NB
    pltpu.make_async_copy(rows_a, out_hbm.at[pl.ds(last, _NB)], sem_wa).wait()
    pltpu.make_async_copy(rows_b, out_hbm.at[pl.ds(last + _NB, _NB)], sem_wb).wait()


def kernel(x, table):
    return _embed_gather(x, table)
